# baseline (speedup 1.0000x reference)
import functools

import jax
import jax.numpy as jnp
from jax import lax
from jax.experimental import pallas as pl
from jax.experimental.pallas import tpu as pltpu
from jax.experimental.pallas import tpu_sc as plsc

_NH = 8
_EPS = 1e-5


def _attn_mean_body(x_ref, w_ref, b_ref, wo_ref, bo_ref, o_ref, *, hd, bb):
    s_len = x_ref.shape[1]
    h = x_ref.shape[2]
    x = x_ref[...].reshape(bb * s_len, h)
    qkv_all = jnp.dot(x, w_ref[...],
                      preferred_element_type=jnp.float32) + b_ref[...]
    sqrt_d = jnp.sqrt(jnp.float32(hd))
    for i in range(bb):
        qkv = qkv_all[i * s_len:(i + 1) * s_len]
        parts = []
        for n in range(_NH):
            q = qkv[:, n * hd:(n + 1) * hd]
            k = qkv[:, h + n * hd:h + (n + 1) * hd]
            v = qkv[:, 2 * h + n * hd:2 * h + (n + 1) * hd]
            s = lax.dot_general(q, k, (((1,), (1,)), ((), ())),
                                preferred_element_type=jnp.float32) / sqrt_d
            m = jnp.max(s, axis=1, keepdims=True)
            e = jnp.exp(s - m)
            r = jnp.sum(e, axis=1, keepdims=True)
            parts.append(jnp.dot(e, v, preferred_element_type=jnp.float32) / r)
        ao = jnp.concatenate(parts, axis=1)
        attn_out = jnp.dot(ao, wo_ref[...],
                           preferred_element_type=jnp.float32) + bo_ref[...]
        o_ref[i] = jnp.mean(attn_out, axis=0, keepdims=True)


def _ln(x, g, b):
    m = jnp.mean(x, axis=-1, keepdims=True)
    v = jnp.mean((x - m) ** 2, axis=-1, keepdims=True)
    return (x - m) * lax.rsqrt(v + _EPS) * g + b


def _gelu(x):
    return x * 0.5 * (1.0 + lax.erf(x * (2.0 ** -0.5)))


def _head_body(seq_ref, cls_ref,
               f1w_ref, f1b_ref, g1_ref, b1_ref,
               f2w_ref, f2b_ref, g2_ref, b2_ref,
               cw_ref, cb_ref, g3_ref, b3_ref,
               r1w_ref, r1b_ref, g4_ref, b4_ref,
               r2w_ref, r2b_ref,
               d1w_ref, d1b_ref, g5_ref, b5_ref,
               d2w_ref, d2b_ref,
               e0w_ref, e0b_ref, e1w_ref, e1b_ref,
               gp_ref, gpt_ref, al0_ref, al1_ref, dom_ref):
    dot = lambda a, b: jnp.dot(a, b, preferred_element_type=jnp.float32)
    seq = seq_ref[...]
    cls = cls_ref[...]
    f = _gelu(_ln(dot(seq, f1w_ref[...]) + f1b_ref[...], g1_ref[...], b1_ref[...]))
    f = _gelu(_ln(dot(f, f2w_ref[...]) + f2b_ref[...], g2_ref[...], b2_ref[...]))
    ctx = _gelu(_ln(dot(f, cw_ref[...]) + cb_ref[...], g3_ref[...], b3_ref[...]))
    r = _gelu(_ln(dot(ctx, r1w_ref[...]) + r1b_ref[...], g4_ref[...], b4_ref[...]))
    gl = dot(r, r2w_ref[...]) + r2b_ref[...]
    gm = jnp.max(gl, axis=1, keepdims=True)
    ge = jnp.exp(gl - gm)
    gp = ge / jnp.sum(ge, axis=1, keepdims=True)
    gp_ref[...] = gp
    gpt_ref[...] = gp.T

    d = _gelu(_ln(dot(cls, d1w_ref[...]) + d1b_ref[...], g5_ref[...], b5_ref[...]))
    dom_ref[...] = dot(d, d2w_ref[...]) + d2b_ref[...]

    al0_ref[...] = (dot(cls, e0w_ref[...]) + e0b_ref[...]).T
    al1_ref[...] = (dot(cls, e1w_ref[...]) + e1b_ref[...]).T


_SC_L = 16


def _route_sc_body(gp_ref, a0_ref, a1_ref, fin_ref, e0_ref, e1_ref,
                   gp_v, a0_v, a1_v, fin_v, e0_v, e1_v):
    n_e, n_tok = gp_ref.shape
    wid = lax.axis_index("s") * 2 + lax.axis_index("c")

    @pl.when(wid == 0)
    def _():
        pltpu.sync_copy(gp_ref, gp_v)
        pltpu.sync_copy(a0_ref, a0_v)
        pltpu.sync_copy(a1_ref, a1_v)
        for c in range(n_tok // _SC_L):
            sl = pl.ds(c * _SC_L, _SC_L)
            g = [gp_v[e, sl] for e in range(n_e)]
            m1 = g[0]
            for e in range(1, n_e):
                m1 = jnp.maximum(m1, g[e])
            i1 = jnp.full((_SC_L,), float(n_e), jnp.float32)
            for e in range(n_e - 1, -1, -1):
                i1 = jnp.where(g[e] == m1, jnp.float32(e), i1)
            g2 = [jnp.where(i1 == jnp.float32(e), jnp.float32(-1.0), g[e])
                  for e in range(n_e)]
            m2 = g2[0]
            for e in range(1, n_e):
                m2 = jnp.maximum(m2, g2[e])
            i2 = jnp.full((_SC_L,), float(n_e), jnp.float32)
            for e in range(n_e - 1, -1, -1):
                i2 = jnp.where(g2[e] == m2, jnp.float32(e), i2)
            den = m1 + m2
            w1 = m1 / den
            w2 = m2 / den
            z = jnp.zeros((_SC_L,), jnp.float32)
            g1_0 = z; g2_0 = z; g1_1 = z; g2_1 = z
            for e in range(n_e):
                sel1 = i1 == jnp.float32(e)
                sel2 = i2 == jnp.float32(e)
                a0e = a0_v[e, sl]
                a1e = a1_v[e, sl]
                g1_0 = jnp.where(sel1, a0e, g1_0)
                g2_0 = jnp.where(sel2, a0e, g2_0)
                g1_1 = jnp.where(sel1, a1e, g1_1)
                g2_1 = jnp.where(sel2, a1e, g2_1)
                both = sel1 | sel2
                e0_v[e, sl] = jnp.where(both, a0e, z)
                e1_v[e, sl] = jnp.where(both, a1e, z)
            fin_v[0, sl] = w1 * g1_0 + w2 * g2_0
            fin_v[1, sl] = w1 * g1_1 + w2 * g2_1
        pltpu.sync_copy(fin_v, fin_ref)
        pltpu.sync_copy(e0_v, e0_ref)
        pltpu.sync_copy(e1_v, e1_ref)


def kernel(hidden_state, in_proj_w, in_proj_b, out_proj_w, out_proj_b,
           fn1_w, fn1_b, ln1_g, ln1_b, fn2_w, fn2_b, ln2_g, ln2_b,
           ctx_w, ctx_b, ln3_g, ln3_b, rh1_w, rh1_b, ln4_g, ln4_b,
           rh2_w, rh2_b, exp_w, exp_b, dh1_w, dh1_b, ln5_g, ln5_b,
           dh2_w, dh2_b):
    b, s, h = hidden_state.shape
    hd = h // _NH
    e_num, l_num, _ = exp_w.shape

    bb = 4
    seq_repr = pl.pallas_call(
        functools.partial(_attn_mean_body, hd=hd, bb=bb),
        grid=(b // bb,),
        in_specs=[
            pl.BlockSpec((bb, s, h), lambda i: (i, 0, 0)),
            pl.BlockSpec((h, 3 * h), lambda i: (0, 0)),
            pl.BlockSpec((1, 3 * h), lambda i: (0, 0)),
            pl.BlockSpec((h, h), lambda i: (0, 0)),
            pl.BlockSpec((1, h), lambda i: (0, 0)),
        ],
        out_specs=pl.BlockSpec((bb, 1, h), lambda i: (i, 0, 0)),
        out_shape=jax.ShapeDtypeStruct((b, 1, h), jnp.float32),
        compiler_params=pltpu.CompilerParams(
            dimension_semantics=("arbitrary",)),
    )(hidden_state, in_proj_w.T, in_proj_b.reshape(1, 3 * h),
      out_proj_w.T, out_proj_b.reshape(1, h))
    seq_repr = seq_repr.reshape(b, h)

    cls = hidden_state[:, 0, :]
    row = lambda t: t.reshape(1, -1)
    outs = pl.pallas_call(
        _head_body,
        out_shape=[
            jax.ShapeDtypeStruct((b, e_num), jnp.float32),
            jax.ShapeDtypeStruct((e_num, b), jnp.float32),
            jax.ShapeDtypeStruct((e_num, b), jnp.float32),
            jax.ShapeDtypeStruct((e_num, b), jnp.float32),
            jax.ShapeDtypeStruct((b, dh2_w.shape[0]), jnp.float32),
        ],
    )(seq_repr, cls,
      fn1_w.T, row(fn1_b), row(ln1_g), row(ln1_b),
      fn2_w.T, row(fn2_b), row(ln2_g), row(ln2_b),
      ctx_w.T, row(ctx_b), row(ln3_g), row(ln3_b),
      rh1_w.T, row(rh1_b), row(ln4_g), row(ln4_b),
      rh2_w.T, row(rh2_b),
      dh1_w.T, row(dh1_b), row(ln5_g), row(ln5_b),
      dh2_w.T, row(dh2_b),
      exp_w[:, 0, :].T, row(exp_b[:, 0]),
      exp_w[:, 1, :].T, row(exp_b[:, 1]))

    gating_probs, gpT, al0T, al1T, domain_logits = outs

    route = pl.kernel(
        _route_sc_body,
        mesh=plsc.VectorSubcoreMesh(core_axis_name="c", subcore_axis_name="s"),
        out_type=[
            jax.ShapeDtypeStruct((l_num, b), jnp.float32),
            jax.ShapeDtypeStruct((e_num, b), jnp.float32),
            jax.ShapeDtypeStruct((e_num, b), jnp.float32),
        ],
        scratch_types=[
            pltpu.VMEM((e_num, b), jnp.float32),
            pltpu.VMEM((e_num, b), jnp.float32),
            pltpu.VMEM((e_num, b), jnp.float32),
            pltpu.VMEM((l_num, b), jnp.float32),
            pltpu.VMEM((e_num, b), jnp.float32),
            pltpu.VMEM((e_num, b), jnp.float32),
        ],
    )
    finT, e0T, e1T = route(gpT, al0T, al1T)
    final_logits = finT.T
    expert_logits = jnp.stack([e0T.T, e1T.T], axis=-1)
    return (final_logits, gating_probs, expert_logits, domain_logits)


# speedup vs baseline: 1.0334x; 1.0334x over previous
import functools

import jax
import jax.numpy as jnp
from jax import lax
from jax.experimental import pallas as pl
from jax.experimental.pallas import tpu as pltpu
from jax.experimental.pallas import tpu_sc as plsc

_NH = 8
_EPS = 1e-5


def _attn_mean_body(x_ref, w_ref, b_ref, wo_ref, bo_ref, o_ref, *, hd, bb):
    s_len = x_ref.shape[1]
    h = x_ref.shape[2]
    x = x_ref[...].reshape(bb * s_len, h)
    qkv_all = jnp.dot(x, w_ref[...],
                      preferred_element_type=jnp.float32) + b_ref[...]
    sqrt_d = jnp.sqrt(jnp.float32(hd))
    for i in range(bb):
        qkv = qkv_all[i * s_len:(i + 1) * s_len]
        parts = []
        for n in range(_NH):
            q = qkv[:, n * hd:(n + 1) * hd]
            k = qkv[:, h + n * hd:h + (n + 1) * hd]
            v = qkv[:, 2 * h + n * hd:2 * h + (n + 1) * hd]
            s = lax.dot_general(q, k, (((1,), (1,)), ((), ())),
                                preferred_element_type=jnp.float32) / sqrt_d
            m = jnp.max(s, axis=1, keepdims=True)
            e = jnp.exp(s - m)
            r = jnp.sum(e, axis=1, keepdims=True)
            parts.append(jnp.dot(e, v, preferred_element_type=jnp.float32) / r)
        ao = jnp.concatenate(parts, axis=1)
        attn_out = jnp.dot(ao, wo_ref[...],
                           preferred_element_type=jnp.float32) + bo_ref[...]
        o_ref[i] = jnp.mean(attn_out, axis=0, keepdims=True)


def _ln(x, g, b):
    m = jnp.mean(x, axis=-1, keepdims=True)
    v = jnp.mean((x - m) ** 2, axis=-1, keepdims=True)
    return (x - m) * lax.rsqrt(v + _EPS) * g + b


def _gelu(x):
    return x * 0.5 * (1.0 + lax.erf(x * (2.0 ** -0.5)))


def _head_body(seq_ref, cls_ref,
               f1w_ref, f1b_ref, g1_ref, b1_ref,
               f2w_ref, f2b_ref, g2_ref, b2_ref,
               cw_ref, cb_ref, g3_ref, b3_ref,
               r1w_ref, r1b_ref, g4_ref, b4_ref,
               r2w_ref, r2b_ref,
               d1w_ref, d1b_ref, g5_ref, b5_ref,
               d2w_ref, d2b_ref,
               e0w_ref, e0b_ref, e1w_ref, e1b_ref,
               gp_ref, gpt_ref, al0_ref, al1_ref, dom_ref):
    dot = lambda a, b: jnp.dot(a, b, preferred_element_type=jnp.float32)
    seq = seq_ref[...]
    cls = cls_ref[...]
    f = _gelu(_ln(dot(seq, f1w_ref[...]) + f1b_ref[...], g1_ref[...], b1_ref[...]))
    f = _gelu(_ln(dot(f, f2w_ref[...]) + f2b_ref[...], g2_ref[...], b2_ref[...]))
    ctx = _gelu(_ln(dot(f, cw_ref[...]) + cb_ref[...], g3_ref[...], b3_ref[...]))
    r = _gelu(_ln(dot(ctx, r1w_ref[...]) + r1b_ref[...], g4_ref[...], b4_ref[...]))
    gl = dot(r, r2w_ref[...]) + r2b_ref[...]
    gm = jnp.max(gl, axis=1, keepdims=True)
    ge = jnp.exp(gl - gm)
    gp = ge / jnp.sum(ge, axis=1, keepdims=True)
    gp_ref[...] = gp
    gpt_ref[...] = gp.T

    d = _gelu(_ln(dot(cls, d1w_ref[...]) + d1b_ref[...], g5_ref[...], b5_ref[...]))
    dom_ref[...] = dot(d, d2w_ref[...]) + d2b_ref[...]

    al0_ref[...] = (dot(cls, e0w_ref[...]) + e0b_ref[...]).T
    al1_ref[...] = (dot(cls, e1w_ref[...]) + e1b_ref[...]).T


_SC_L = 16


def _route_sc_body(gp_ref, a0_ref, a1_ref, fin_ref, e0_ref, e1_ref,
                   gp_v, a0_v, a1_v, fin_v, e0_v, e1_v):
    n_e, n_tok = gp_ref.shape
    wid = lax.axis_index("s") * 2 + lax.axis_index("c")

    @pl.when(wid == 0)
    def _():
        pltpu.sync_copy(gp_ref, gp_v)
        pltpu.sync_copy(a0_ref, a0_v)
        pltpu.sync_copy(a1_ref, a1_v)
        for c in range(n_tok // _SC_L):
            sl = pl.ds(c * _SC_L, _SC_L)
            g = [gp_v[e, sl] for e in range(n_e)]
            m1 = g[0]
            for e in range(1, n_e):
                m1 = jnp.maximum(m1, g[e])
            i1 = jnp.full((_SC_L,), float(n_e), jnp.float32)
            for e in range(n_e - 1, -1, -1):
                i1 = jnp.where(g[e] == m1, jnp.float32(e), i1)
            g2 = [jnp.where(i1 == jnp.float32(e), jnp.float32(-1.0), g[e])
                  for e in range(n_e)]
            m2 = g2[0]
            for e in range(1, n_e):
                m2 = jnp.maximum(m2, g2[e])
            i2 = jnp.full((_SC_L,), float(n_e), jnp.float32)
            for e in range(n_e - 1, -1, -1):
                i2 = jnp.where(g2[e] == m2, jnp.float32(e), i2)
            den = m1 + m2
            w1 = m1 / den
            w2 = m2 / den
            z = jnp.zeros((_SC_L,), jnp.float32)
            g1_0 = z; g2_0 = z; g1_1 = z; g2_1 = z
            for e in range(n_e):
                sel1 = i1 == jnp.float32(e)
                sel2 = i2 == jnp.float32(e)
                a0e = a0_v[e, sl]
                a1e = a1_v[e, sl]
                g1_0 = jnp.where(sel1, a0e, g1_0)
                g2_0 = jnp.where(sel2, a0e, g2_0)
                g1_1 = jnp.where(sel1, a1e, g1_1)
                g2_1 = jnp.where(sel2, a1e, g2_1)
                both = sel1 | sel2
                e0_v[e, sl] = jnp.where(both, a0e, z)
                e1_v[e, sl] = jnp.where(both, a1e, z)
            fin_v[0, sl] = w1 * g1_0 + w2 * g2_0
            fin_v[1, sl] = w1 * g1_1 + w2 * g2_1
        pltpu.sync_copy(fin_v, fin_ref)
        pltpu.sync_copy(e0_v, e0_ref)
        pltpu.sync_copy(e1_v, e1_ref)


def kernel(hidden_state, in_proj_w, in_proj_b, out_proj_w, out_proj_b,
           fn1_w, fn1_b, ln1_g, ln1_b, fn2_w, fn2_b, ln2_g, ln2_b,
           ctx_w, ctx_b, ln3_g, ln3_b, rh1_w, rh1_b, ln4_g, ln4_b,
           rh2_w, rh2_b, exp_w, exp_b, dh1_w, dh1_b, ln5_g, ln5_b,
           dh2_w, dh2_b):
    b, s, h = hidden_state.shape
    hd = h // _NH
    e_num, l_num, _ = exp_w.shape

    bb = 2
    seq_repr = pl.pallas_call(
        functools.partial(_attn_mean_body, hd=hd, bb=bb),
        grid=(b // bb,),
        in_specs=[
            pl.BlockSpec((bb, s, h), lambda i: (i, 0, 0)),
            pl.BlockSpec((h, 3 * h), lambda i: (0, 0)),
            pl.BlockSpec((1, 3 * h), lambda i: (0, 0)),
            pl.BlockSpec((h, h), lambda i: (0, 0)),
            pl.BlockSpec((1, h), lambda i: (0, 0)),
        ],
        out_specs=pl.BlockSpec((bb, 1, h), lambda i: (i, 0, 0)),
        out_shape=jax.ShapeDtypeStruct((b, 1, h), jnp.float32),
        compiler_params=pltpu.CompilerParams(
            dimension_semantics=("arbitrary",)),
    )(hidden_state, in_proj_w.T, in_proj_b.reshape(1, 3 * h),
      out_proj_w.T, out_proj_b.reshape(1, h))
    seq_repr = seq_repr.reshape(b, h)

    cls = hidden_state[:, 0, :]
    row = lambda t: t.reshape(1, -1)
    outs = pl.pallas_call(
        _head_body,
        out_shape=[
            jax.ShapeDtypeStruct((b, e_num), jnp.float32),
            jax.ShapeDtypeStruct((e_num, b), jnp.float32),
            jax.ShapeDtypeStruct((e_num, b), jnp.float32),
            jax.ShapeDtypeStruct((e_num, b), jnp.float32),
            jax.ShapeDtypeStruct((b, dh2_w.shape[0]), jnp.float32),
        ],
    )(seq_repr, cls,
      fn1_w.T, row(fn1_b), row(ln1_g), row(ln1_b),
      fn2_w.T, row(fn2_b), row(ln2_g), row(ln2_b),
      ctx_w.T, row(ctx_b), row(ln3_g), row(ln3_b),
      rh1_w.T, row(rh1_b), row(ln4_g), row(ln4_b),
      rh2_w.T, row(rh2_b),
      dh1_w.T, row(dh1_b), row(ln5_g), row(ln5_b),
      dh2_w.T, row(dh2_b),
      exp_w[:, 0, :].T, row(exp_b[:, 0]),
      exp_w[:, 1, :].T, row(exp_b[:, 1]))

    gating_probs, gpT, al0T, al1T, domain_logits = outs

    route = pl.kernel(
        _route_sc_body,
        mesh=plsc.VectorSubcoreMesh(core_axis_name="c", subcore_axis_name="s"),
        out_type=[
            jax.ShapeDtypeStruct((l_num, b), jnp.float32),
            jax.ShapeDtypeStruct((e_num, b), jnp.float32),
            jax.ShapeDtypeStruct((e_num, b), jnp.float32),
        ],
        scratch_types=[
            pltpu.VMEM((e_num, b), jnp.float32),
            pltpu.VMEM((e_num, b), jnp.float32),
            pltpu.VMEM((e_num, b), jnp.float32),
            pltpu.VMEM((l_num, b), jnp.float32),
            pltpu.VMEM((e_num, b), jnp.float32),
            pltpu.VMEM((e_num, b), jnp.float32),
        ],
    )
    finT, e0T, e1T = route(gpT, al0T, al1T)
    final_logits = finT.T
    expert_logits = jnp.stack([e0T.T, e1T.T], axis=-1)
    return (final_logits, gating_probs, expert_logits, domain_logits)
